# Initial kernel scaffold; baseline (speedup 1.0000x reference)
import jax
import jax.numpy as jnp
from jax.experimental import pallas as pl


def kernel(query, value, rand_matrix):
    raise NotImplementedError("write your pallas kernel here")


# speedup vs baseline: 22.3311x; 22.3311x over previous
import functools
import math

import jax
import jax.numpy as jnp
from jax import lax
from jax.experimental import pallas as pl
from jax.experimental.pallas import tpu as pltpu
from jax.experimental.pallas import tpu_sc as plsc

D = 64
R = 4
BL = 64
B = 4
L = 8192
NB = L // BL
NH = 128
BR = B * R

NC = 2
NS = 16

_SC_MESH = dict(core_axis_name="c", subcore_axis_name="s", num_cores=NC,
                num_subcores=NS)


LBLK = 2048


def _hash_body(q_ref, rm_ref, h_ref):
    q = q_ref[0]
    qn = q / jnp.maximum(
        jnp.sqrt(jnp.sum(q * q, axis=1, keepdims=True)), 1e-12)
    rm = rm_ref[0]
    rmn = rm / jnp.sqrt(jnp.sum(rm * rm, axis=0, keepdims=True))
    xr = jnp.dot(qn, rmn, preferred_element_type=jnp.float32)
    rows = []
    for r in range(R):
        sub = xr[:, r * 64:(r + 1) * 64]
        cat = jnp.concatenate([sub, -sub], axis=1)
        rows.append(jnp.argmax(cat, axis=1).astype(jnp.int32)[None, :])
    h_ref[0] = jnp.concatenate(rows, axis=0)


def _hash_stage(q, rm2):
    return pl.pallas_call(
        _hash_body,
        grid=(B, L // LBLK),
        in_specs=[
            pl.BlockSpec((1, LBLK, D), lambda b, j: (b, j, 0)),
            pl.BlockSpec((1, D, R * NH // 2), lambda b, j: (b, 0, 0)),
        ],
        out_specs=pl.BlockSpec((1, R, LBLK), lambda b, j: (b, 0, j)),
        out_shape=jax.ShapeDtypeStruct((B, R, L), jnp.int32),
    )(q, rm2)


SEG = L // 16


def _sort_body(h_hbm, perm_hbm, sh_hbm, h_v, perm_v, sh_v, cnt2, spref,
               off_v, tmp_v):
    wid = lax.axis_index("c") * NS + lax.axis_index("s")

    @pl.when(wid < BR)
    def _():
        iota = lax.iota(jnp.int32, 16)
        zero = jnp.zeros((16,), jnp.int32)
        row0 = pl.multiple_of(wid * L, L)
        pltpu.sync_copy(h_hbm.at[pl.ds(row0, L)], h_v)
        for l in range(16):
            for c in range(NH // 16):
                cnt2[l, pl.ds(c * 16, 16)] = zero

        def hist(i, carry):
            hv = plsc.load_gather(h_v, [iota * SEG + i])
            c = plsc.load_gather(cnt2, [iota, hv])
            plsc.store_scatter(cnt2, [iota, hv], c + 1)
            return carry

        lax.fori_loop(0, SEG, hist, 0)

        for c in range(NH // 16):
            acc = zero
            for l in range(16):
                row = cnt2[l, pl.ds(c * 16, 16)]
                spref[l, pl.ds(c * 16, 16)] = acc
                acc = acc + row
            off_v[pl.ds(c * 16, 16)] = acc

        carry = zero
        fifteen = jnp.full((16,), 15, jnp.int32)
        for c in range(NH // 16):
            x = off_v[pl.ds(c * 16, 16)]
            s = x
            for shbit in (1, 2, 4, 8):
                tmp_v[pl.ds(0, 16)] = zero
                tmp_v[pl.ds(shbit, 16)] = s
                s = s + tmp_v[pl.ds(0, 16)]
            off_v[pl.ds(c * 16, 16)] = s - x + carry
            tmp_v[pl.ds(0, 16)] = s
            carry = carry + plsc.load_gather(tmp_v, [fifteen])

        for c in range(NH // 16):
            base = off_v[pl.ds(c * 16, 16)]
            for l in range(16):
                spref[l, pl.ds(c * 16, 16)] = (
                    spref[l, pl.ds(c * 16, 16)] + base)
                cnt2[l, pl.ds(c * 16, 16)] = zero

        def place(i, carry):
            pos = iota * SEG + i
            hv = plsc.load_gather(h_v, [pos])
            sp = plsc.load_gather(spref, [iota, hv])
            cc = plsc.load_gather(cnt2, [iota, hv])
            dest = sp + cc
            plsc.store_scatter(cnt2, [iota, hv], cc + 1)
            plsc.store_scatter(perm_v, [dest], pos)
            plsc.store_scatter(sh_v, [dest], hv)
            return carry

        lax.fori_loop(0, SEG, place, 0)
        pltpu.sync_copy(perm_v, perm_hbm.at[pl.ds(row0, L)])
        pltpu.sync_copy(sh_v, sh_hbm.at[pl.ds(row0, L)])


def _sort_stage(h):
    f = pl.kernel(
        _sort_body,
        out_type=(jax.ShapeDtypeStruct((BR * L,), jnp.int32),
                  jax.ShapeDtypeStruct((BR * L,), jnp.int32)),
        mesh=plsc.VectorSubcoreMesh(**_SC_MESH),
        compiler_params=pltpu.CompilerParams(needs_layout_passes=False),
        scratch_types=[
            pltpu.VMEM((L,), jnp.int32),
            pltpu.VMEM((L,), jnp.int32),
            pltpu.VMEM((L,), jnp.int32),
            pltpu.VMEM((16, NH), jnp.int32),
            pltpu.VMEM((16, NH), jnp.int32),
            pltpu.VMEM((NH,), jnp.int32),
            pltpu.VMEM((48,), jnp.int32),
        ],
    )
    return f(h)


NCHUNK = 32


def _gather_body(qv_hbm, perm3_hbm, qvs_hbm, pbuf, idx2, rows0, rows1, sem):
    wid = lax.axis_index("c") * NS + lax.axis_index("s")
    pair = wid // 2
    half = wid % 2
    b = pair // R
    base = half * (L // 2)

    pltpu.sync_copy(
        perm3_hbm.at[pair,
                     pl.ds(pl.multiple_of(half * NCHUNK, NCHUNK), NCHUNK)],
        pbuf)
    off = jnp.full((16,), b * L, jnp.int32)
    for j in range(NCHUNK):
        for k in range(8):
            idx2[j, pl.ds(k * 16, 16)] = pbuf[j, pl.ds(k * 16, 16)] + off
    bufs = (rows0, rows1)
    cp0 = pltpu.async_copy(qv_hbm.at[idx2.at[0]], rows0, sem)
    for j in range(NCHUNK):
        cp_next = None
        if j + 1 < NCHUNK:
            cp_next = pltpu.async_copy(qv_hbm.at[idx2.at[j + 1]],
                                       bufs[(j + 1) % 2], sem)
        if j == 0:
            cp0.wait()
        else:
            cp_prev.wait()
        pltpu.sync_copy(
            bufs[j % 2],
            qvs_hbm.at[pair, pl.ds(pl.multiple_of(base + j * 128, 128),
                                   128)])
        cp_prev = cp_next


def _gather_stage(qv, perm3):
    f = pl.kernel(
        _gather_body,
        out_type=jax.ShapeDtypeStruct((BR, L, 2 * D), jnp.float32),
        mesh=plsc.VectorSubcoreMesh(**_SC_MESH),
        compiler_params=pltpu.CompilerParams(needs_layout_passes=False),
        scratch_types=[
            pltpu.VMEM((NCHUNK, 128), jnp.int32),
            pltpu.VMEM((NCHUNK, 128), jnp.int32),
            pltpu.VMEM((128, 2 * D), jnp.float32),
            pltpu.VMEM((128, 2 * D), jnp.float32),
            pltpu.SemaphoreType.DMA,
        ],
    )
    return f(qv, perm3)


CHQ = 256
KW = CHQ + BL


def _attn_body(qvc_ref, qvh_ref, shc_ref, shh_ref, numer_ref, s_ref):
    j = pl.program_id(1)
    qc = qvc_ref[0, :, :D]
    qh = qvh_ref[0, :, :D]
    k_src = jnp.concatenate([qh, qc], axis=0)
    kn = k_src / jnp.maximum(
        jnp.sqrt(jnp.sum(k_src * k_src, axis=1, keepdims=True)), 1e-12)
    qk = lax.dot_general(qc, kn, (((1,), (1,)), ((), ())),
                         preferred_element_type=jnp.float32)
    qk = qk * (1.0 / math.sqrt(D))
    shq = shc_ref[0, 0]
    shk = jnp.concatenate([shh_ref[0, 0], shq])
    row = lax.broadcasted_iota(jnp.int32, (CHQ, KW), 0)
    col = lax.broadcasted_iota(jnp.int32, (CHQ, KW), 1)
    qb = row // BL
    kb = col // BL
    band = (kb == qb) | (kb == qb + 1)
    selfm = col == row + BL
    hmask = shq[:, None] == shk[None, :]
    u = jnp.where(band & hmask & (~selfm), jnp.exp(qk), 0.0)
    v_src = jnp.concatenate([qvh_ref[0, :, D:], qvc_ref[0, :, D:]],
                            axis=0)
    numer_ref[0] = jnp.dot(u, v_src, preferred_element_type=jnp.float32)
    part = jnp.sum(u)
    lane = lax.broadcasted_iota(jnp.int32, (1, 128), 1)
    prev = jnp.where(j == 0, jnp.zeros((1, 128), jnp.float32), s_ref[0])
    s_ref[0] = prev + jnp.where(lane == 0, part, 0.0)


def _attn_stage(qvs, sh_cur3, sh_halo3):
    nj = L // CHQ
    bpc = CHQ // BL
    return pl.pallas_call(
        _attn_body,
        grid=(BR, nj),
        in_specs=[
            pl.BlockSpec((1, CHQ, 2 * D), lambda br, j: (br, j, 0)),
            pl.BlockSpec((1, BL, 2 * D),
                         lambda br, j: (br, (j * bpc - 1) % NB, 0)),
            pl.BlockSpec((1, 1, CHQ), lambda br, j: (br * nj + j, 0, 0)),
            pl.BlockSpec((1, 1, BL),
                         lambda br, j: (br * NB + (j * bpc - 1) % NB, 0, 0)),
        ],
        out_specs=[
            pl.BlockSpec((1, CHQ, D), lambda br, j: (br, j, 0)),
            pl.BlockSpec((1, 1, 128), lambda br, j: (br, 0, 0)),
        ],
        out_shape=[
            jax.ShapeDtypeStruct((BR, L, D), jnp.float32),
            jax.ShapeDtypeStruct((BR, 1, 128), jnp.float32),
        ],
    )(qvs, qvs, sh_cur3, sh_halo3)


def _combine_body(numer2_hbm, perm3_hbm, s_hbm, out3_hbm,
                  accum_sh, nbuf, nbuf2, idx2, pbuf, sbuf):
    c = lax.axis_index("c")
    s = lax.axis_index("s")
    pair = s // 2
    half = s % 2
    b_loc = pair // R
    r = pair % R
    br = (2 * c + b_loc) * R + r

    zero16 = jnp.zeros((16,), jnp.float32)
    for rr in range(128):
        for c4 in range(D // 16):
            nbuf[rr, pl.ds(c4 * 16, 16)] = zero16
    for k in range(8):
        pltpu.sync_copy(
            nbuf,
            accum_sh.at[pl.ds(pl.multiple_of(s * 1024 + k * 128, 128), 128)])
    plsc.subcore_barrier()

    pltpu.sync_copy(s_hbm.at[pl.ds(pl.multiple_of(br * 128, 128), 128)],
                    sbuf)
    zeros_i = jnp.zeros((16,), jnp.int32)
    invv = 1.0 / plsc.load_gather(sbuf, [zeros_i])

    pltpu.sync_copy(
        perm3_hbm.at[br,
                     pl.ds(pl.multiple_of(half * NCHUNK, NCHUNK), NCHUNK)],
        pbuf)
    off = jnp.full((16,), b_loc * L, jnp.int32)
    for j in range(NCHUNK):
        for k in range(8):
            idx2[j, pl.ds(k * 16, 16)] = pbuf[j, pl.ds(k * 16, 16)] + off

    for j in range(NCHUNK):
        pltpu.sync_copy(
            numer2_hbm.at[br, pl.ds(pl.multiple_of(half * 2048 + j * 64, 64),
                                    64)],
            nbuf2)

        def rearr(i, carry):
            for c4 in range(D // 16):
                x = nbuf2[i, pl.ds(c4 * 16, 16)]
                nbuf[2 * i, pl.ds(c4 * 16, 16)] = x * invv
                y = nbuf2[i, pl.ds(D + c4 * 16, 16)]
                nbuf[2 * i + 1, pl.ds(c4 * 16, 16)] = y * invv
            return carry

        lax.fori_loop(0, 64, rearr, 0)
        pltpu.sync_copy(nbuf, accum_sh.at[idx2.at[j]], add=True)

    plsc.subcore_barrier()
    for k in range(8):
        pltpu.sync_copy(
            accum_sh.at[pl.ds(pl.multiple_of(s * 1024 + k * 128, 128), 128)],
            nbuf)

        def rearr2(i, carry):
            for c4 in range(D // 16):
                nbuf2[i, pl.ds(c4 * 16, 16)] = nbuf[2 * i, pl.ds(c4 * 16, 16)]
                nbuf2[i, pl.ds(D + c4 * 16, 16)] = (
                    nbuf[2 * i + 1, pl.ds(c4 * 16, 16)])
            return carry

        lax.fori_loop(0, 64, rearr2, 0)
        pltpu.sync_copy(
            nbuf2,
            out3_hbm.at[pl.ds(
                pl.multiple_of(c * L + s * 512 + k * 64, 64), 64)])


def _combine_stage(numer2, perm3, s_arr):
    f = pl.kernel(
        _combine_body,
        out_type=jax.ShapeDtypeStruct((B * L * D // 128, 128), jnp.float32),
        mesh=plsc.VectorSubcoreMesh(**_SC_MESH),
        compiler_params=pltpu.CompilerParams(needs_layout_passes=False),
        scratch_types=[
            pltpu.VMEM_SHARED((2 * L, D), jnp.float32),
            pltpu.VMEM((128, D), jnp.float32),
            pltpu.VMEM((64, 2 * D), jnp.float32),
            pltpu.VMEM((NCHUNK, 128), jnp.int32),
            pltpu.VMEM((NCHUNK, 128), jnp.int32),
            pltpu.VMEM((128,), jnp.float32),
        ],
    )
    return f(numer2, perm3, s_arr)


def kernel(query, value, rand_matrix):
    rm2 = rand_matrix.reshape(B, D, R * NH // 2)
    h = _hash_stage(query, rm2)
    h2 = h.reshape(BR * L)
    perm, sh = _sort_stage(h2)
    perm3 = perm.reshape(BR, L // 128, 128)
    qv = jnp.concatenate([query, value], axis=-1).reshape(B * L, 2 * D)
    qvs = _gather_stage(qv, perm3)
    sh_cur3 = sh.reshape(BR * (L // CHQ), 1, CHQ)
    sh_halo3 = sh.reshape(BR * NB, 1, BL)
    numer, s_arr = _attn_stage(qvs, sh_cur3, sh_halo3)
    numer2 = numer.reshape(BR, L * D // 128, 128)
    s2 = s_arr.reshape(BR * 128)
    out3 = _combine_stage(numer2, perm3, s2)
    return out3.reshape(B, L, D)


# speedup vs baseline: 23.3185x; 1.0442x over previous
import functools
import math

import jax
import jax.numpy as jnp
from jax import lax
from jax.experimental import pallas as pl
from jax.experimental.pallas import tpu as pltpu
from jax.experimental.pallas import tpu_sc as plsc

D = 64
R = 4
BL = 64
B = 4
L = 8192
NB = L // BL
NH = 128
BR = B * R

NC = 2
NS = 16

_SC_MESH = dict(core_axis_name="c", subcore_axis_name="s", num_cores=NC,
                num_subcores=NS)


LBLK = 2048


def _hash_body(q_ref, rm_ref, h_ref):
    q = q_ref[0]
    qn = q / jnp.maximum(
        jnp.sqrt(jnp.sum(q * q, axis=1, keepdims=True)), 1e-12)
    rm = rm_ref[0]
    rmn = rm / jnp.sqrt(jnp.sum(rm * rm, axis=0, keepdims=True))
    xr = jnp.dot(qn, rmn, preferred_element_type=jnp.float32)
    rows = []
    for r in range(R):
        sub = xr[:, r * 64:(r + 1) * 64]
        cat = jnp.concatenate([sub, -sub], axis=1)
        rows.append(jnp.argmax(cat, axis=1).astype(jnp.int32)[None, :])
    h_ref[0] = jnp.concatenate(rows, axis=0)


def _hash_stage(q, rm2):
    return pl.pallas_call(
        _hash_body,
        grid=(B, L // LBLK),
        in_specs=[
            pl.BlockSpec((1, LBLK, D), lambda b, j: (b, j, 0)),
            pl.BlockSpec((1, D, R * NH // 2), lambda b, j: (b, 0, 0)),
        ],
        out_specs=pl.BlockSpec((1, R, LBLK), lambda b, j: (b, 0, j)),
        out_shape=jax.ShapeDtypeStruct((B, R, L), jnp.int32),
    )(q, rm2)


SEG = L // 16


def _sort_body(h_hbm, perm_hbm, sh_hbm, h_v, perm_v, sh_v, cnt2, spref,
               off_v, tmp_v):
    wid = lax.axis_index("c") * NS + lax.axis_index("s")

    @pl.when(wid < BR)
    def _():
        iota = lax.iota(jnp.int32, 16)
        zero = jnp.zeros((16,), jnp.int32)
        row0 = pl.multiple_of(wid * L, L)
        pltpu.sync_copy(h_hbm.at[pl.ds(row0, L)], h_v)
        for l in range(16):
            for c in range(NH // 16):
                cnt2[l, pl.ds(c * 16, 16)] = zero

        def hist(i, carry):
            hv = plsc.load_gather(h_v, [iota * SEG + i])
            c = plsc.load_gather(cnt2, [iota, hv])
            plsc.store_scatter(cnt2, [iota, hv], c + 1)
            return carry

        lax.fori_loop(0, SEG, hist, 0)

        for c in range(NH // 16):
            acc = zero
            for l in range(16):
                row = cnt2[l, pl.ds(c * 16, 16)]
                spref[l, pl.ds(c * 16, 16)] = acc
                acc = acc + row
            off_v[pl.ds(c * 16, 16)] = acc

        carry = zero
        fifteen = jnp.full((16,), 15, jnp.int32)
        for c in range(NH // 16):
            x = off_v[pl.ds(c * 16, 16)]
            s = x
            for shbit in (1, 2, 4, 8):
                tmp_v[pl.ds(0, 16)] = zero
                tmp_v[pl.ds(shbit, 16)] = s
                s = s + tmp_v[pl.ds(0, 16)]
            off_v[pl.ds(c * 16, 16)] = s - x + carry
            tmp_v[pl.ds(0, 16)] = s
            carry = carry + plsc.load_gather(tmp_v, [fifteen])

        for c in range(NH // 16):
            base = off_v[pl.ds(c * 16, 16)]
            for l in range(16):
                spref[l, pl.ds(c * 16, 16)] = (
                    spref[l, pl.ds(c * 16, 16)] + base)
                cnt2[l, pl.ds(c * 16, 16)] = zero

        def place(i, carry):
            pos = iota * SEG + i
            hv = plsc.load_gather(h_v, [pos])
            sp = plsc.load_gather(spref, [iota, hv])
            cc = plsc.load_gather(cnt2, [iota, hv])
            dest = sp + cc
            plsc.store_scatter(cnt2, [iota, hv], cc + 1)
            plsc.store_scatter(perm_v, [dest], pos)
            plsc.store_scatter(sh_v, [dest], hv)
            return carry

        lax.fori_loop(0, SEG, place, 0)
        pltpu.sync_copy(perm_v, perm_hbm.at[pl.ds(row0, L)])
        pltpu.sync_copy(sh_v, sh_hbm.at[pl.ds(row0, L)])


def _sort_stage(h):
    f = pl.kernel(
        _sort_body,
        out_type=(jax.ShapeDtypeStruct((BR * L,), jnp.int32),
                  jax.ShapeDtypeStruct((BR * L,), jnp.int32)),
        mesh=plsc.VectorSubcoreMesh(**_SC_MESH),
        compiler_params=pltpu.CompilerParams(needs_layout_passes=False),
        scratch_types=[
            pltpu.VMEM((L,), jnp.int32),
            pltpu.VMEM((L,), jnp.int32),
            pltpu.VMEM((L,), jnp.int32),
            pltpu.VMEM((16, NH), jnp.int32),
            pltpu.VMEM((16, NH), jnp.int32),
            pltpu.VMEM((NH,), jnp.int32),
            pltpu.VMEM((48,), jnp.int32),
        ],
    )
    return f(h)


NCHUNK = 32


def _gather_body(qv_hbm, perm3_hbm, qvs_hbm, pbuf, idx2, rows0, rows1, sem):
    wid = lax.axis_index("c") * NS + lax.axis_index("s")
    pair = wid // 2
    half = wid % 2
    b = pair // R
    base = half * (L // 2)

    pltpu.sync_copy(
        perm3_hbm.at[pair,
                     pl.ds(pl.multiple_of(half * NCHUNK, NCHUNK), NCHUNK)],
        pbuf)
    off = jnp.full((16,), b * L, jnp.int32)
    for j in range(NCHUNK):
        for k in range(8):
            idx2[j, pl.ds(k * 16, 16)] = pbuf[j, pl.ds(k * 16, 16)] + off
    bufs = (rows0, rows1)
    cp0 = pltpu.async_copy(qv_hbm.at[idx2.at[0]], rows0, sem)
    for j in range(NCHUNK):
        cp_next = None
        if j + 1 < NCHUNK:
            cp_next = pltpu.async_copy(qv_hbm.at[idx2.at[j + 1]],
                                       bufs[(j + 1) % 2], sem)
        if j == 0:
            cp0.wait()
        else:
            cp_prev.wait()
        pltpu.sync_copy(
            bufs[j % 2],
            qvs_hbm.at[pair, pl.ds(pl.multiple_of(base + j * 128, 128),
                                   128)])
        cp_prev = cp_next


def _gather_stage(qv, perm3):
    f = pl.kernel(
        _gather_body,
        out_type=jax.ShapeDtypeStruct((BR, L, 2 * D), jnp.float32),
        mesh=plsc.VectorSubcoreMesh(**_SC_MESH),
        compiler_params=pltpu.CompilerParams(needs_layout_passes=False),
        scratch_types=[
            pltpu.VMEM((NCHUNK, 128), jnp.int32),
            pltpu.VMEM((NCHUNK, 128), jnp.int32),
            pltpu.VMEM((128, 2 * D), jnp.float32),
            pltpu.VMEM((128, 2 * D), jnp.float32),
            pltpu.SemaphoreType.DMA,
        ],
    )
    return f(qv, perm3)


CHQ = 256
KW = CHQ + BL


def _attn_body(qvc_ref, qvh_ref, shc_ref, shh_ref, numer_ref, s_ref):
    j = pl.program_id(1)
    qc = qvc_ref[0, :, :D]
    qh = qvh_ref[0, :, :D]
    k_src = jnp.concatenate([qh, qc], axis=0)
    kn = k_src / jnp.maximum(
        jnp.sqrt(jnp.sum(k_src * k_src, axis=1, keepdims=True)), 1e-12)
    qk = lax.dot_general(qc.astype(jnp.bfloat16), kn.astype(jnp.bfloat16),
                         (((1,), (1,)), ((), ())),
                         preferred_element_type=jnp.float32)
    qk = qk * (1.0 / math.sqrt(D))
    shq = shc_ref[0, 0]
    shk = jnp.concatenate([shh_ref[0, 0], shq])
    row = lax.broadcasted_iota(jnp.int32, (CHQ, KW), 0)
    col = lax.broadcasted_iota(jnp.int32, (CHQ, KW), 1)
    qb = row // BL
    kb = col // BL
    band = (kb == qb) | (kb == qb + 1)
    selfm = col == row + BL
    hmask = shq[:, None] == shk[None, :]
    u = jnp.where(band & hmask & (~selfm), jnp.exp(qk), 0.0)
    v_src = jnp.concatenate([qvh_ref[0, :, D:], qvc_ref[0, :, D:]],
                            axis=0)
    numer_ref[0] = jnp.dot(u.astype(jnp.bfloat16),
                           v_src.astype(jnp.bfloat16),
                           preferred_element_type=jnp.float32)
    part = jnp.sum(u)
    lane = lax.broadcasted_iota(jnp.int32, (1, 128), 1)
    prev = jnp.where(j == 0, jnp.zeros((1, 128), jnp.float32), s_ref[0])
    s_ref[0] = prev + jnp.where(lane == 0, part, 0.0)


def _attn_stage(qvs, sh_cur3, sh_halo3):
    nj = L // CHQ
    bpc = CHQ // BL
    return pl.pallas_call(
        _attn_body,
        grid=(BR, nj),
        in_specs=[
            pl.BlockSpec((1, CHQ, 2 * D), lambda br, j: (br, j, 0)),
            pl.BlockSpec((1, BL, 2 * D),
                         lambda br, j: (br, (j * bpc - 1) % NB, 0)),
            pl.BlockSpec((1, 1, CHQ), lambda br, j: (br * nj + j, 0, 0)),
            pl.BlockSpec((1, 1, BL),
                         lambda br, j: (br * NB + (j * bpc - 1) % NB, 0, 0)),
        ],
        out_specs=[
            pl.BlockSpec((1, CHQ, D), lambda br, j: (br, j, 0)),
            pl.BlockSpec((1, 1, 128), lambda br, j: (br, 0, 0)),
        ],
        out_shape=[
            jax.ShapeDtypeStruct((BR, L, D), jnp.float32),
            jax.ShapeDtypeStruct((BR, 1, 128), jnp.float32),
        ],
    )(qvs, qvs, sh_cur3, sh_halo3)


def _combine_body(numer2_hbm, perm3_hbm, s_hbm, out3_hbm,
                  accum_sh, rb0, rb1, wb0, wb1, idx2, pbuf, sbuf,
                  sem_r, sem_w):
    c = lax.axis_index("c")
    s = lax.axis_index("s")
    pair = s // 2
    half = s % 2
    b_loc = pair // R
    r = pair % R
    br = (2 * c + b_loc) * R + r

    zero16 = jnp.zeros((16,), jnp.float32)

    def zrow(i, carry):
        for c4 in range(D // 16):
            wb0[i, pl.ds(c4 * 16, 16)] = zero16
        return carry

    lax.fori_loop(0, 128, zrow, 0)
    for k in range(8):
        pltpu.async_copy(
            wb0,
            accum_sh.at[pl.ds(pl.multiple_of(s * 1024 + k * 128, 128), 128)],
            sem_w)
    for k in range(8):
        pltpu.make_async_copy(
            wb0,
            accum_sh.at[pl.ds(pl.multiple_of(s * 1024 + k * 128, 128), 128)],
            sem_w).wait()
    plsc.subcore_barrier()

    pltpu.sync_copy(s_hbm.at[pl.ds(pl.multiple_of(br * 128, 128), 128)],
                    sbuf)
    zeros_i = jnp.zeros((16,), jnp.int32)
    invv = 1.0 / plsc.load_gather(sbuf, [zeros_i])

    pltpu.sync_copy(
        perm3_hbm.at[br,
                     pl.ds(pl.multiple_of(half * NCHUNK, NCHUNK), NCHUNK)],
        pbuf)
    off = jnp.full((16,), b_loc * L, jnp.int32)

    def mkidx(jj, carry):
        for k in range(8):
            idx2[jj, pl.ds(k * 16, 16)] = pbuf[jj, pl.ds(k * 16, 16)] + off
        return carry

    lax.fori_loop(0, NCHUNK, mkidx, 0)

    rbufs = (rb0, rb1)

    def chunk_src(j):
        return numer2_hbm.at[
            br, pl.ds(pl.multiple_of(half * 2048 + j * 64, 64), 64)]

    pltpu.async_copy(chunk_src(0), rb0, sem_r)
    pltpu.async_copy(chunk_src(1), rb1, sem_r)

    def main_body(t, carry):
        j0 = t * 2
        pltpu.make_async_copy(chunk_src(j0), rb0, sem_r).wait()

        def rearr0(i, carry2):
            for r4 in range(4):
                row = i * 4 + r4
                for c4 in range(D // 16):
                    x = rb0[row, pl.ds(c4 * 16, 16)]
                    wb0[2 * row, pl.ds(c4 * 16, 16)] = x * invv
                    y = rb0[row, pl.ds(D + c4 * 16, 16)]
                    wb0[2 * row + 1, pl.ds(c4 * 16, 16)] = y * invv
            return carry2

        lax.fori_loop(0, 16, rearr0, 0)
        pltpu.async_copy(chunk_src(jnp.minimum(j0 + 2, NCHUNK - 1)), rb0,
                         sem_r)
        pltpu.sync_copy(wb0, accum_sh.at[idx2.at[j0]], add=True)

        pltpu.make_async_copy(chunk_src(j0 + 1), rb1, sem_r).wait()

        def rearr1(i, carry2):
            for r4 in range(4):
                row = i * 4 + r4
                for c4 in range(D // 16):
                    x = rb1[row, pl.ds(c4 * 16, 16)]
                    wb1[2 * row, pl.ds(c4 * 16, 16)] = x * invv
                    y = rb1[row, pl.ds(D + c4 * 16, 16)]
                    wb1[2 * row + 1, pl.ds(c4 * 16, 16)] = y * invv
            return carry2

        lax.fori_loop(0, 16, rearr1, 0)
        pltpu.async_copy(chunk_src(jnp.minimum(j0 + 3, NCHUNK - 1)), rb1,
                         sem_r)
        pltpu.sync_copy(wb1, accum_sh.at[idx2.at[j0 + 1]], add=True)
        return carry

    lax.fori_loop(0, NCHUNK // 2, main_body, 0)
    pltpu.make_async_copy(chunk_src(NCHUNK - 1), rb0, sem_r).wait()
    pltpu.make_async_copy(chunk_src(NCHUNK - 1), rb1, sem_r).wait()
    plsc.subcore_barrier()

    def out_body(k, carry):
        pltpu.sync_copy(
            accum_sh.at[pl.ds(pl.multiple_of(s * 1024 + k * 128, 128), 128)],
            wb0)

        def rearr2(i, carry2):
            for r4 in range(4):
                row = i * 4 + r4
                for c4 in range(D // 16):
                    rb0[row, pl.ds(c4 * 16, 16)] = (
                        wb0[2 * row, pl.ds(c4 * 16, 16)])
                    rb0[row, pl.ds(D + c4 * 16, 16)] = (
                        wb0[2 * row + 1, pl.ds(c4 * 16, 16)])
            return carry2

        lax.fori_loop(0, 16, rearr2, 0)
        pltpu.sync_copy(
            rb0,
            out3_hbm.at[pl.ds(
                pl.multiple_of(c * L + s * 512 + k * 64, 64), 64)])
        return carry

    lax.fori_loop(0, 8, out_body, 0)


def _combine_stage(numer2, perm3, s_arr):
    f = pl.kernel(
        _combine_body,
        out_type=jax.ShapeDtypeStruct((B * L * D // 128, 128), jnp.float32),
        mesh=plsc.VectorSubcoreMesh(**_SC_MESH),
        compiler_params=pltpu.CompilerParams(needs_layout_passes=False),
        scratch_types=[
            pltpu.VMEM_SHARED((2 * L, D), jnp.float32),
            pltpu.VMEM((64, 2 * D), jnp.float32),
            pltpu.VMEM((64, 2 * D), jnp.float32),
            pltpu.VMEM((128, D), jnp.float32),
            pltpu.VMEM((128, D), jnp.float32),
            pltpu.VMEM((NCHUNK, 128), jnp.int32),
            pltpu.VMEM((NCHUNK, 128), jnp.int32),
            pltpu.VMEM((128,), jnp.float32),
            pltpu.SemaphoreType.DMA,
            pltpu.SemaphoreType.DMA,
        ],
    )
    return f(numer2, perm3, s_arr)


def kernel(query, value, rand_matrix):
    rm2 = rand_matrix.reshape(B, D, R * NH // 2)
    h = _hash_stage(query, rm2)
    h2 = h.reshape(BR * L)
    perm, sh = _sort_stage(h2)
    perm3 = perm.reshape(BR, L // 128, 128)
    qv = jnp.concatenate([query, value], axis=-1).reshape(B * L, 2 * D)
    qvs = _gather_stage(qv, perm3)
    sh_cur3 = sh.reshape(BR * (L // CHQ), 1, CHQ)
    sh_halo3 = sh.reshape(BR * NB, 1, BL)
    numer, s_arr = _attn_stage(qvs, sh_cur3, sh_halo3)
    numer2 = numer.reshape(BR, L * D // 128, 128)
    s2 = s_arr.reshape(BR * 128)
    out3 = _combine_stage(numer2, perm3, s2)
    return out3.reshape(B, L, D)


# speedup vs baseline: 27.1080x; 1.1625x over previous
import functools
import math

import jax
import jax.numpy as jnp
from jax import lax
from jax.experimental import pallas as pl
from jax.experimental.pallas import tpu as pltpu
from jax.experimental.pallas import tpu_sc as plsc

D = 64
R = 4
BL = 64
B = 4
L = 8192
NB = L // BL
NH = 128
BR = B * R

NC = 2
NS = 16

_SC_MESH = dict(core_axis_name="c", subcore_axis_name="s", num_cores=NC,
                num_subcores=NS)


LBLK = 2048


def _hash_body(q_ref, v_ref, rm_ref, h_ref, qv_ref):
    q = q_ref[0]
    qn = q / jnp.maximum(
        jnp.sqrt(jnp.sum(q * q, axis=1, keepdims=True)), 1e-12)
    rm = rm_ref[0]
    rmn = rm / jnp.sqrt(jnp.sum(rm * rm, axis=0, keepdims=True))
    xr = jnp.dot(qn, rmn, preferred_element_type=jnp.float32)
    rows = []
    for r in range(R):
        sub = xr[:, r * 64:(r + 1) * 64]
        cat = jnp.concatenate([sub, -sub], axis=1)
        rows.append(jnp.argmax(cat, axis=1).astype(jnp.int32)[None, :])
    h_ref[0] = jnp.concatenate(rows, axis=0)
    qv_ref[0] = jnp.concatenate([q, v_ref[0]], axis=1)


def _hash_stage(q, v, rm2):
    return pl.pallas_call(
        _hash_body,
        grid=(B, L // LBLK),
        in_specs=[
            pl.BlockSpec((1, LBLK, D), lambda b, j: (b, j, 0)),
            pl.BlockSpec((1, LBLK, D), lambda b, j: (b, j, 0)),
            pl.BlockSpec((1, D, R * NH // 2), lambda b, j: (b, 0, 0)),
        ],
        out_specs=[
            pl.BlockSpec((1, R, LBLK), lambda b, j: (b, 0, j)),
            pl.BlockSpec((1, LBLK, 2 * D), lambda b, j: (b, j, 0)),
        ],
        out_shape=[
            jax.ShapeDtypeStruct((B, R, L), jnp.int32),
            jax.ShapeDtypeStruct((B, L, 2 * D), jnp.float32),
        ],
    )(q, v, rm2)


SEG = L // 16


def _sort_body(h_hbm, perm_hbm, sh_hbm, h_v, perm_v, sh_v, cnt2, spref,
               off_v, tmp_v):
    wid = lax.axis_index("c") * NS + lax.axis_index("s")

    @pl.when(wid < BR)
    def _():
        iota = lax.iota(jnp.int32, 16)
        zero = jnp.zeros((16,), jnp.int32)
        row0 = pl.multiple_of(wid * L, L)
        pltpu.sync_copy(h_hbm.at[pl.ds(row0, L)], h_v)
        for l in range(16):
            for c in range(NH // 16):
                cnt2[l, pl.ds(c * 16, 16)] = zero

        def hist(i, carry):
            hv = plsc.load_gather(h_v, [iota * SEG + i])
            c = plsc.load_gather(cnt2, [iota, hv])
            plsc.store_scatter(cnt2, [iota, hv], c + 1)
            return carry

        lax.fori_loop(0, SEG, hist, 0)

        for c in range(NH // 16):
            acc = zero
            for l in range(16):
                row = cnt2[l, pl.ds(c * 16, 16)]
                spref[l, pl.ds(c * 16, 16)] = acc
                acc = acc + row
            off_v[pl.ds(c * 16, 16)] = acc

        carry = zero
        fifteen = jnp.full((16,), 15, jnp.int32)
        for c in range(NH // 16):
            x = off_v[pl.ds(c * 16, 16)]
            s = x
            for shbit in (1, 2, 4, 8):
                tmp_v[pl.ds(0, 16)] = zero
                tmp_v[pl.ds(shbit, 16)] = s
                s = s + tmp_v[pl.ds(0, 16)]
            off_v[pl.ds(c * 16, 16)] = s - x + carry
            tmp_v[pl.ds(0, 16)] = s
            carry = carry + plsc.load_gather(tmp_v, [fifteen])

        for c in range(NH // 16):
            base = off_v[pl.ds(c * 16, 16)]
            for l in range(16):
                spref[l, pl.ds(c * 16, 16)] = (
                    spref[l, pl.ds(c * 16, 16)] + base)
                cnt2[l, pl.ds(c * 16, 16)] = zero

        def place(i, carry):
            pos = iota * SEG + i
            hv = plsc.load_gather(h_v, [pos])
            sp = plsc.load_gather(spref, [iota, hv])
            cc = plsc.load_gather(cnt2, [iota, hv])
            dest = sp + cc
            plsc.store_scatter(cnt2, [iota, hv], cc + 1)
            plsc.store_scatter(perm_v, [dest], pos)
            plsc.store_scatter(sh_v, [dest], hv)
            return carry

        lax.fori_loop(0, SEG, place, 0)
        pltpu.sync_copy(perm_v, perm_hbm.at[pl.ds(row0, L)])
        pltpu.sync_copy(sh_v, sh_hbm.at[pl.ds(row0, L)])


def _sort_stage(h):
    f = pl.kernel(
        _sort_body,
        out_type=(jax.ShapeDtypeStruct((BR * L,), jnp.int32),
                  jax.ShapeDtypeStruct((BR * L,), jnp.int32)),
        mesh=plsc.VectorSubcoreMesh(**_SC_MESH),
        compiler_params=pltpu.CompilerParams(needs_layout_passes=False),
        scratch_types=[
            pltpu.VMEM((L,), jnp.int32),
            pltpu.VMEM((L,), jnp.int32),
            pltpu.VMEM((L,), jnp.int32),
            pltpu.VMEM((16, NH), jnp.int32),
            pltpu.VMEM((16, NH), jnp.int32),
            pltpu.VMEM((NH,), jnp.int32),
            pltpu.VMEM((48,), jnp.int32),
        ],
    )
    return f(h)


NCHUNK = 32


def _gather_body(qv_hbm, perm3_hbm, qvs_hbm, pbuf, idx2, rows0, rows1, sem):
    wid = lax.axis_index("c") * NS + lax.axis_index("s")
    pair = wid // 2
    half = wid % 2
    b = pair // R
    base = half * (L // 2)

    pltpu.sync_copy(
        perm3_hbm.at[pair,
                     pl.ds(pl.multiple_of(half * NCHUNK, NCHUNK), NCHUNK)],
        pbuf)
    off = jnp.full((16,), b * L, jnp.int32)
    for j in range(NCHUNK):
        for k in range(8):
            idx2[j, pl.ds(k * 16, 16)] = pbuf[j, pl.ds(k * 16, 16)] + off
    bufs = (rows0, rows1)
    cp0 = pltpu.async_copy(qv_hbm.at[idx2.at[0]], rows0, sem)
    for j in range(NCHUNK):
        cp_next = None
        if j + 1 < NCHUNK:
            cp_next = pltpu.async_copy(qv_hbm.at[idx2.at[j + 1]],
                                       bufs[(j + 1) % 2], sem)
        if j == 0:
            cp0.wait()
        else:
            cp_prev.wait()
        pltpu.sync_copy(
            bufs[j % 2],
            qvs_hbm.at[pair, pl.ds(pl.multiple_of(base + j * 128, 128),
                                   128)])
        cp_prev = cp_next


def _gather_stage(qv, perm3):
    f = pl.kernel(
        _gather_body,
        out_type=jax.ShapeDtypeStruct((BR, L, 2 * D), jnp.float32),
        mesh=plsc.VectorSubcoreMesh(**_SC_MESH),
        compiler_params=pltpu.CompilerParams(needs_layout_passes=False),
        scratch_types=[
            pltpu.VMEM((NCHUNK, 128), jnp.int32),
            pltpu.VMEM((NCHUNK, 128), jnp.int32),
            pltpu.VMEM((128, 2 * D), jnp.float32),
            pltpu.VMEM((128, 2 * D), jnp.float32),
            pltpu.SemaphoreType.DMA,
        ],
    )
    return f(qv, perm3)


CHQ = 256
KW = CHQ + BL


def _attn_body(qvc_ref, qvh_ref, shc_ref, shh_ref, mb_ref, numer_ref, s_ref):
    j = pl.program_id(1)
    qc = qvc_ref[0, :, :D]
    qh = qvh_ref[0, :, :D]
    k_src = jnp.concatenate([qh, qc], axis=0)
    kn = k_src / jnp.maximum(
        jnp.sqrt(jnp.sum(k_src * k_src, axis=1, keepdims=True)), 1e-12)
    qk = lax.dot_general(qc.astype(jnp.bfloat16), kn.astype(jnp.bfloat16),
                         (((1,), (1,)), ((), ())),
                         preferred_element_type=jnp.float32)
    qk = qk * (1.0 / math.sqrt(D)) + mb_ref[0]
    shq = shc_ref[0, 0]
    shk = jnp.concatenate([shh_ref[0, 0], shq])
    hmask = shq[:, None] == shk[None, :]
    u = jnp.where(hmask, jnp.exp(qk), 0.0)
    v_src = jnp.concatenate([qvh_ref[0, :, D:], qvc_ref[0, :, D:]],
                            axis=0)
    numer_ref[0, :, :D] = jnp.dot(u.astype(jnp.bfloat16),
                                  v_src.astype(jnp.bfloat16),
                                  preferred_element_type=jnp.float32)
    part = jnp.sum(u)
    lane = lax.broadcasted_iota(jnp.int32, (1, 128), 1)
    prev = jnp.where(j == 0, jnp.zeros((1, 128), jnp.float32), s_ref[0])
    s_ref[0] = prev + jnp.where(lane == 0, part, 0.0)


def _attn_stage(qvs, sh_cur3, sh_halo3, mbias):
    nj = L // CHQ
    bpc = CHQ // BL
    return pl.pallas_call(
        _attn_body,
        grid=(BR, nj),
        in_specs=[
            pl.BlockSpec((1, CHQ, 2 * D), lambda br, j: (br, j, 0)),
            pl.BlockSpec((1, BL, 2 * D),
                         lambda br, j: (br, (j * bpc - 1) % NB, 0)),
            pl.BlockSpec((1, 1, CHQ), lambda br, j: (br * nj + j, 0, 0)),
            pl.BlockSpec((1, 1, BL),
                         lambda br, j: (br * NB + (j * bpc - 1) % NB, 0, 0)),
            pl.BlockSpec((1, CHQ, KW), lambda br, j: (0, 0, 0)),
        ],
        out_specs=[
            pl.BlockSpec((1, CHQ, 128), lambda br, j: (br, j, 0)),
            pl.BlockSpec((1, 1, 128), lambda br, j: (br, 0, 0)),
        ],
        out_shape=[
            jax.ShapeDtypeStruct((BR, L, 128), jnp.float32),
            jax.ShapeDtypeStruct((BR, 1, 128), jnp.float32),
        ],
    )(qvs, qvs, sh_cur3, sh_halo3, mbias)


def _combine_body(numer2_hbm, perm3_hbm, s_hbm, out3_hbm,
                  accum_sh, rb0, rb1, wb0, idx2, sbuf,
                  sem_r, sem_w):
    c = lax.axis_index("c")
    s = lax.axis_index("s")
    pair = s // 2
    half = s % 2
    b_loc = pair // R
    r = pair % R
    br = (2 * c + b_loc) * R + r

    zero16 = jnp.zeros((16,), jnp.float32)

    def zrow(i, carry):
        for c4 in range(D // 16):
            wb0[i, pl.ds(c4 * 16, 16)] = zero16
        return carry

    lax.fori_loop(0, 128, zrow, 0)
    for k in range(8):
        pltpu.async_copy(
            wb0,
            accum_sh.at[pl.ds(pl.multiple_of(s * 1024 + k * 128, 128), 128)],
            sem_w)
    for k in range(8):
        pltpu.make_async_copy(
            wb0,
            accum_sh.at[pl.ds(pl.multiple_of(s * 1024 + k * 128, 128), 128)],
            sem_w).wait()
    plsc.subcore_barrier()

    pltpu.sync_copy(s_hbm.at[pl.ds(pl.multiple_of(br * 128, 128), 128)],
                    sbuf)
    zeros_i = jnp.zeros((16,), jnp.int32)
    invv = 1.0 / plsc.load_gather(sbuf, [zeros_i])

    pltpu.sync_copy(
        perm3_hbm.at[br,
                     pl.ds(pl.multiple_of(half * NCHUNK, NCHUNK), NCHUNK)],
        idx2)
    off = jnp.full((16,), b_loc * L, jnp.int32)

    def mkidx(jj, carry):
        for k in range(8):
            idx2[jj, pl.ds(k * 16, 16)] = idx2[jj, pl.ds(k * 16, 16)] + off
        return carry

    lax.fori_loop(0, NCHUNK, mkidx, 0)

    rbufs = (rb0, rb1)

    def chunk_src(j):
        return numer2_hbm.at[
            br, pl.ds(pl.multiple_of(half * (L // 2) + j * 128, 128), 128)]

    pltpu.async_copy(chunk_src(0), rb0, sem_r)
    pltpu.async_copy(chunk_src(1), rb1, sem_r)

    def main_body(t, carry):
        j0 = t * 2
        pltpu.make_async_copy(chunk_src(j0), rb0, sem_r).wait()

        def rearr0(i, carry2):
            for r4 in range(4):
                row = i * 4 + r4
                for c4 in range(D // 16):
                    x = rb0[row, pl.ds(c4 * 16, 16)]
                    wb0[row, pl.ds(c4 * 16, 16)] = x * invv
            return carry2

        lax.fori_loop(0, 32, rearr0, 0)
        pltpu.async_copy(chunk_src(jnp.minimum(j0 + 2, NCHUNK - 1)), rb0,
                         sem_r)
        pltpu.sync_copy(wb0, accum_sh.at[idx2.at[j0]], add=True)

        pltpu.make_async_copy(chunk_src(j0 + 1), rb1, sem_r).wait()

        def rearr1(i, carry2):
            for r4 in range(4):
                row = i * 4 + r4
                for c4 in range(D // 16):
                    x = rb1[row, pl.ds(c4 * 16, 16)]
                    wb0[row, pl.ds(c4 * 16, 16)] = x * invv
            return carry2

        lax.fori_loop(0, 32, rearr1, 0)
        pltpu.async_copy(chunk_src(jnp.minimum(j0 + 3, NCHUNK - 1)), rb1,
                         sem_r)
        pltpu.sync_copy(wb0, accum_sh.at[idx2.at[j0 + 1]], add=True)
        return carry

    lax.fori_loop(0, NCHUNK // 2, main_body, 0)
    pltpu.make_async_copy(chunk_src(NCHUNK - 1), rb0, sem_r).wait()
    pltpu.make_async_copy(chunk_src(NCHUNK - 1), rb1, sem_r).wait()
    plsc.subcore_barrier()

    def out_body(k, carry):
        pltpu.sync_copy(
            accum_sh.at[pl.ds(pl.multiple_of(s * 1024 + k * 128, 128), 128)],
            wb0)

        def rearr2(i, carry2):
            for r4 in range(4):
                row = i * 4 + r4
                for c4 in range(D // 16):
                    rb0[row, pl.ds(c4 * 16, 16)] = (
                        wb0[2 * row, pl.ds(c4 * 16, 16)])
                    rb0[row, pl.ds(D + c4 * 16, 16)] = (
                        wb0[2 * row + 1, pl.ds(c4 * 16, 16)])
            return carry2

        lax.fori_loop(0, 16, rearr2, 0)
        pltpu.sync_copy(
            rb0.at[pl.ds(0, 64)],
            out3_hbm.at[pl.ds(
                pl.multiple_of(c * L + s * 512 + k * 64, 64), 64)])
        return carry

    lax.fori_loop(0, 8, out_body, 0)


def _combine_stage(numer2, perm3, s_arr):
    f = pl.kernel(
        _combine_body,
        out_type=jax.ShapeDtypeStruct((B * L * D // 128, 128), jnp.float32),
        mesh=plsc.VectorSubcoreMesh(**_SC_MESH),
        compiler_params=pltpu.CompilerParams(needs_layout_passes=False),
        scratch_types=[
            pltpu.VMEM_SHARED((2 * L, D), jnp.float32),
            pltpu.VMEM((128, 128), jnp.float32),
            pltpu.VMEM((128, 128), jnp.float32),
            pltpu.VMEM((128, D), jnp.float32),
            pltpu.VMEM((NCHUNK, 128), jnp.int32),
            pltpu.VMEM((128,), jnp.float32),
            pltpu.SemaphoreType.DMA,
            pltpu.SemaphoreType.DMA,
        ],
    )
    return f(numer2, perm3, s_arr)


def _make_mbias():
    r_ = jnp.arange(CHQ, dtype=jnp.int32)[:, None]
    c_ = jnp.arange(KW, dtype=jnp.int32)[None, :]
    qb = r_ // BL
    kb = c_ // BL
    band = (kb == qb) | (kb == qb + 1)
    selfm = c_ == r_ + BL
    return jnp.where(band & (~selfm), 0.0, -1e9).astype(jnp.float32)[None]


def kernel(query, value, rand_matrix):
    rm2 = rand_matrix.reshape(B, D, R * NH // 2)
    h, qv3 = _hash_stage(query, value, rm2)
    h2 = h.reshape(BR * L)
    perm, sh = _sort_stage(h2)
    perm3 = perm.reshape(BR, L // 128, 128)
    qvs = _gather_stage(qv3.reshape(B * L, 2 * D), perm3)
    sh_cur3 = sh.reshape(BR * (L // CHQ), 1, CHQ)
    sh_halo3 = sh.reshape(BR * NB, 1, BL)
    numer, s_arr = _attn_stage(qvs, sh_cur3, sh_halo3, _make_mbias())
    s2 = s_arr.reshape(BR * 128)
    out3 = _combine_stage(numer, perm3, s2)
    return out3.reshape(B, L, D)


# speedup vs baseline: 32.8704x; 1.2126x over previous
import functools
import math

import jax
import jax.numpy as jnp
from jax import lax
from jax.experimental import pallas as pl
from jax.experimental.pallas import tpu as pltpu
from jax.experimental.pallas import tpu_sc as plsc

D = 64
R = 4
BL = 64
B = 4
L = 8192
NB = L // BL
NH = 128
BR = B * R

NC = 2
NS = 16

_SC_MESH = dict(core_axis_name="c", subcore_axis_name="s", num_cores=NC,
                num_subcores=NS)


LBLK = 2048


def _hash_body(q_ref, v_ref, rm_ref, h_ref, qv_ref):
    q = q_ref[0]
    qn = q / jnp.maximum(
        jnp.sqrt(jnp.sum(q * q, axis=1, keepdims=True)), 1e-12)
    rm = rm_ref[0]
    rmn = rm / jnp.sqrt(jnp.sum(rm * rm, axis=0, keepdims=True))
    xr = jnp.dot(qn, rmn, preferred_element_type=jnp.float32)
    rows = []
    for r in range(R):
        sub = xr[:, r * 64:(r + 1) * 64]
        cat = jnp.concatenate([sub, -sub], axis=1)
        rows.append(jnp.argmax(cat, axis=1).astype(jnp.int32)[None, :])
    h_ref[0] = jnp.concatenate(rows, axis=0)
    qv_ref[0] = jnp.concatenate([q, v_ref[0]], axis=1)


def _hash_stage(q, v, rm2):
    return pl.pallas_call(
        _hash_body,
        grid=(B, L // LBLK),
        in_specs=[
            pl.BlockSpec((1, LBLK, D), lambda b, j: (b, j, 0)),
            pl.BlockSpec((1, LBLK, D), lambda b, j: (b, j, 0)),
            pl.BlockSpec((1, D, R * NH // 2), lambda b, j: (b, 0, 0)),
        ],
        out_specs=[
            pl.BlockSpec((1, R, LBLK), lambda b, j: (b, 0, j)),
            pl.BlockSpec((1, LBLK, 2 * D), lambda b, j: (b, j, 0)),
        ],
        out_shape=[
            jax.ShapeDtypeStruct((B, R, L), jnp.int32),
            jax.ShapeDtypeStruct((B, L, 2 * D), jnp.float32),
        ],
    )(q, v, rm2)


SEG = L // 16


def _sort_body(h_hbm, perm_hbm, sh_hbm, h_v, perm_v, sh_v, cnt2, spref,
               off_v, tmp_v):
    wid = lax.axis_index("c") * NS + lax.axis_index("s")

    @pl.when(wid < BR)
    def _():
        iota = lax.iota(jnp.int32, 16)
        zero = jnp.zeros((16,), jnp.int32)
        row0 = pl.multiple_of(wid * L, L)
        pltpu.sync_copy(h_hbm.at[pl.ds(row0, L)], h_v)
        for l in range(16):
            for c in range(NH // 16):
                cnt2[l, pl.ds(c * 16, 16)] = zero

        def hist(i, carry):
            hv = plsc.load_gather(h_v, [iota * SEG + i])
            c = plsc.load_gather(cnt2, [iota, hv])
            plsc.store_scatter(cnt2, [iota, hv], c + 1)
            return carry

        lax.fori_loop(0, SEG, hist, 0)

        for c in range(NH // 16):
            acc = zero
            for l in range(16):
                row = cnt2[l, pl.ds(c * 16, 16)]
                spref[l, pl.ds(c * 16, 16)] = acc
                acc = acc + row
            off_v[pl.ds(c * 16, 16)] = acc

        carry = zero
        fifteen = jnp.full((16,), 15, jnp.int32)
        for c in range(NH // 16):
            x = off_v[pl.ds(c * 16, 16)]
            s = x
            for shbit in (1, 2, 4, 8):
                tmp_v[pl.ds(0, 16)] = zero
                tmp_v[pl.ds(shbit, 16)] = s
                s = s + tmp_v[pl.ds(0, 16)]
            off_v[pl.ds(c * 16, 16)] = s - x + carry
            tmp_v[pl.ds(0, 16)] = s
            carry = carry + plsc.load_gather(tmp_v, [fifteen])

        for c in range(NH // 16):
            base = off_v[pl.ds(c * 16, 16)]
            for l in range(16):
                spref[l, pl.ds(c * 16, 16)] = (
                    spref[l, pl.ds(c * 16, 16)] + base)
                cnt2[l, pl.ds(c * 16, 16)] = zero

        def place(i, carry):
            pos = iota * SEG + i
            hv = plsc.load_gather(h_v, [pos])
            sp = plsc.load_gather(spref, [iota, hv])
            cc = plsc.load_gather(cnt2, [iota, hv])
            dest = sp + cc
            plsc.store_scatter(cnt2, [iota, hv], cc + 1)
            plsc.store_scatter(perm_v, [dest], pos)
            plsc.store_scatter(sh_v, [dest], hv)
            return carry

        lax.fori_loop(0, SEG, place, 0)
        pltpu.sync_copy(perm_v, perm_hbm.at[pl.ds(row0, L)])
        pltpu.sync_copy(sh_v, sh_hbm.at[pl.ds(row0, L)])


def _sort_stage(h):
    f = pl.kernel(
        _sort_body,
        out_type=(jax.ShapeDtypeStruct((BR * L,), jnp.int32),
                  jax.ShapeDtypeStruct((BR * L,), jnp.int32)),
        mesh=plsc.VectorSubcoreMesh(**_SC_MESH),
        compiler_params=pltpu.CompilerParams(needs_layout_passes=False),
        scratch_types=[
            pltpu.VMEM((L,), jnp.int32),
            pltpu.VMEM((L,), jnp.int32),
            pltpu.VMEM((L,), jnp.int32),
            pltpu.VMEM((16, NH), jnp.int32),
            pltpu.VMEM((16, NH), jnp.int32),
            pltpu.VMEM((NH,), jnp.int32),
            pltpu.VMEM((48,), jnp.int32),
        ],
    )
    return f(h)


NCHUNK = 32


def _gather_body(qv_hbm, perm3_hbm, qvs_hbm, pbuf, idx2, rows0, rows1, sem):
    wid = lax.axis_index("c") * NS + lax.axis_index("s")
    pair = wid // 2
    half = wid % 2
    b = pair // R
    base = half * (L // 2)

    pltpu.sync_copy(
        perm3_hbm.at[pair,
                     pl.ds(pl.multiple_of(half * NCHUNK, NCHUNK), NCHUNK)],
        pbuf)
    off = jnp.full((16,), b * L, jnp.int32)
    for j in range(NCHUNK):
        for k in range(8):
            idx2[j, pl.ds(k * 16, 16)] = pbuf[j, pl.ds(k * 16, 16)] + off
    bufs = (rows0, rows1)
    cp0 = pltpu.async_copy(qv_hbm.at[idx2.at[0]], rows0, sem)
    for j in range(NCHUNK):
        cp_next = None
        if j + 1 < NCHUNK:
            cp_next = pltpu.async_copy(qv_hbm.at[idx2.at[j + 1]],
                                       bufs[(j + 1) % 2], sem)
        if j == 0:
            cp0.wait()
        else:
            cp_prev.wait()
        pltpu.sync_copy(
            bufs[j % 2],
            qvs_hbm.at[pair, pl.ds(pl.multiple_of(base + j * 128, 128),
                                   128)])
        cp_prev = cp_next


def _gather_stage(qv, perm3):
    f = pl.kernel(
        _gather_body,
        out_type=jax.ShapeDtypeStruct((BR, L, 2 * D), jnp.float32),
        mesh=plsc.VectorSubcoreMesh(**_SC_MESH),
        compiler_params=pltpu.CompilerParams(needs_layout_passes=False),
        scratch_types=[
            pltpu.VMEM((NCHUNK, 128), jnp.int32),
            pltpu.VMEM((NCHUNK, 128), jnp.int32),
            pltpu.VMEM((128, 2 * D), jnp.float32),
            pltpu.VMEM((128, 2 * D), jnp.float32),
            pltpu.SemaphoreType.DMA,
        ],
    )
    return f(qv, perm3)


CHQ = 512
KW = CHQ + BL


def _attn_body(qvc_ref, qvh_ref, shc_ref, shh_ref, mb_ref, numer_ref, s_ref):
    j = pl.program_id(1)
    qc = qvc_ref[0, :, :D]
    qh = qvh_ref[0, :, :D]
    k_src = jnp.concatenate([qh, qc], axis=0)
    kn = k_src / jnp.maximum(
        jnp.sqrt(jnp.sum(k_src * k_src, axis=1, keepdims=True)), 1e-12)
    qk = lax.dot_general(qc.astype(jnp.bfloat16), kn.astype(jnp.bfloat16),
                         (((1,), (1,)), ((), ())),
                         preferred_element_type=jnp.float32)
    qk = qk * (1.0 / math.sqrt(D)) + mb_ref[0]
    shq = shc_ref[0, 0]
    shk = jnp.concatenate([shh_ref[0, 0], shq])
    hmask = shq[:, None] == shk[None, :]
    u = jnp.where(hmask, jnp.exp(qk), 0.0)
    v_src = jnp.concatenate([qvh_ref[0, :, D:], qvc_ref[0, :, D:]],
                            axis=0)
    numer_ref[0, :, :D] = jnp.dot(u.astype(jnp.bfloat16),
                                  v_src.astype(jnp.bfloat16),
                                  preferred_element_type=jnp.float32)
    part = jnp.sum(u)
    lane = lax.broadcasted_iota(jnp.int32, (1, 128), 1)
    prev = jnp.where(j == 0, jnp.zeros((1, 128), jnp.float32), s_ref[0])
    s_ref[0] = prev + jnp.where(lane == 0, part, 0.0)


def _attn_stage(qvs, sh_cur3, sh_halo3, mbias):
    nj = L // CHQ
    bpc = CHQ // BL
    return pl.pallas_call(
        _attn_body,
        grid=(BR, nj),
        in_specs=[
            pl.BlockSpec((1, CHQ, 2 * D), lambda br, j: (br, j, 0)),
            pl.BlockSpec((1, BL, 2 * D),
                         lambda br, j: (br, (j * bpc - 1) % NB, 0)),
            pl.BlockSpec((1, 1, CHQ), lambda br, j: (br * nj + j, 0, 0)),
            pl.BlockSpec((1, 1, BL),
                         lambda br, j: (br * NB + (j * bpc - 1) % NB, 0, 0)),
            pl.BlockSpec((1, CHQ, KW), lambda br, j: (0, 0, 0)),
        ],
        out_specs=[
            pl.BlockSpec((1, CHQ, 128), lambda br, j: (br, j, 0)),
            pl.BlockSpec((1, 1, 128), lambda br, j: (br, 0, 0)),
        ],
        out_shape=[
            jax.ShapeDtypeStruct((BR, L, 128), jnp.float32),
            jax.ShapeDtypeStruct((BR, 1, 128), jnp.float32),
        ],
    )(qvs, qvs, sh_cur3, sh_halo3, mbias)


def _combine_body(numer2_hbm, perm3_hbm, s_hbm, out3_hbm,
                  accum_sh, rb0, rb1, wb0, idx2, sbuf,
                  sem_r, sem_w):
    c = lax.axis_index("c")
    s = lax.axis_index("s")
    pair = s // 2
    half = s % 2
    b_loc = pair // R
    r = pair % R
    br = (2 * c + b_loc) * R + r

    zero16 = jnp.zeros((16,), jnp.float32)

    def zrow(i, carry):
        for c4 in range(D // 16):
            wb0[i, pl.ds(c4 * 16, 16)] = zero16
        return carry

    lax.fori_loop(0, 128, zrow, 0)
    for k in range(8):
        pltpu.async_copy(
            wb0,
            accum_sh.at[pl.ds(pl.multiple_of(s * 1024 + k * 128, 128), 128)],
            sem_w)
    for k in range(8):
        pltpu.make_async_copy(
            wb0,
            accum_sh.at[pl.ds(pl.multiple_of(s * 1024 + k * 128, 128), 128)],
            sem_w).wait()
    plsc.subcore_barrier()

    pltpu.sync_copy(s_hbm.at[pl.ds(pl.multiple_of(br * 128, 128), 128)],
                    sbuf)
    zeros_i = jnp.zeros((16,), jnp.int32)
    invv = 1.0 / plsc.load_gather(sbuf, [zeros_i])

    pltpu.sync_copy(
        perm3_hbm.at[br,
                     pl.ds(pl.multiple_of(half * NCHUNK, NCHUNK), NCHUNK)],
        idx2)
    off = jnp.full((16,), b_loc * L, jnp.int32)

    def mkidx(jj, carry):
        for k in range(8):
            idx2[jj, pl.ds(k * 16, 16)] = idx2[jj, pl.ds(k * 16, 16)] + off
        return carry

    lax.fori_loop(0, NCHUNK, mkidx, 0)

    rbufs = (rb0, rb1)

    def chunk_src(j):
        return numer2_hbm.at[
            br, pl.ds(pl.multiple_of(half * (L // 2) + j * 128, 128), 128)]

    pltpu.async_copy(chunk_src(0), rb0, sem_r)
    pltpu.async_copy(chunk_src(1), rb1, sem_r)

    def main_body(t, carry):
        j0 = t * 2
        pltpu.make_async_copy(chunk_src(j0), rb0, sem_r).wait()

        def rearr0(i, carry2):
            for r4 in range(4):
                row = i * 4 + r4
                for c4 in range(D // 16):
                    x = rb0[row, pl.ds(c4 * 16, 16)]
                    wb0[row, pl.ds(c4 * 16, 16)] = x * invv
            return carry2

        lax.fori_loop(0, 32, rearr0, 0)
        pltpu.async_copy(chunk_src(jnp.minimum(j0 + 2, NCHUNK - 1)), rb0,
                         sem_r)
        pltpu.sync_copy(wb0, accum_sh.at[idx2.at[j0]], add=True)

        pltpu.make_async_copy(chunk_src(j0 + 1), rb1, sem_r).wait()

        def rearr1(i, carry2):
            for r4 in range(4):
                row = i * 4 + r4
                for c4 in range(D // 16):
                    x = rb1[row, pl.ds(c4 * 16, 16)]
                    wb0[row, pl.ds(c4 * 16, 16)] = x * invv
            return carry2

        lax.fori_loop(0, 32, rearr1, 0)
        pltpu.async_copy(chunk_src(jnp.minimum(j0 + 3, NCHUNK - 1)), rb1,
                         sem_r)
        pltpu.sync_copy(wb0, accum_sh.at[idx2.at[j0 + 1]], add=True)
        return carry

    lax.fori_loop(0, NCHUNK // 2, main_body, 0)
    pltpu.make_async_copy(chunk_src(NCHUNK - 1), rb0, sem_r).wait()
    pltpu.make_async_copy(chunk_src(NCHUNK - 1), rb1, sem_r).wait()
    plsc.subcore_barrier()

    def out_body(k, carry):
        pltpu.sync_copy(
            accum_sh.at[pl.ds(pl.multiple_of(s * 1024 + k * 128, 128), 128)],
            wb0)

        def rearr2(i, carry2):
            for r4 in range(4):
                row = i * 4 + r4
                for c4 in range(D // 16):
                    rb0[row, pl.ds(c4 * 16, 16)] = (
                        wb0[2 * row, pl.ds(c4 * 16, 16)])
                    rb0[row, pl.ds(D + c4 * 16, 16)] = (
                        wb0[2 * row + 1, pl.ds(c4 * 16, 16)])
            return carry2

        lax.fori_loop(0, 16, rearr2, 0)
        pltpu.sync_copy(
            rb0.at[pl.ds(0, 64)],
            out3_hbm.at[pl.ds(
                pl.multiple_of(c * L + s * 512 + k * 64, 64), 64)])
        return carry

    lax.fori_loop(0, 8, out_body, 0)


def _combine_stage(numer2, perm3, s_arr):
    f = pl.kernel(
        _combine_body,
        out_type=jax.ShapeDtypeStruct((B * L * D // 128, 128), jnp.float32),
        mesh=plsc.VectorSubcoreMesh(**_SC_MESH),
        compiler_params=pltpu.CompilerParams(needs_layout_passes=False),
        scratch_types=[
            pltpu.VMEM_SHARED((2 * L, D), jnp.float32),
            pltpu.VMEM((128, 128), jnp.float32),
            pltpu.VMEM((128, 128), jnp.float32),
            pltpu.VMEM((128, D), jnp.float32),
            pltpu.VMEM((NCHUNK, 128), jnp.int32),
            pltpu.VMEM((128,), jnp.float32),
            pltpu.SemaphoreType.DMA,
            pltpu.SemaphoreType.DMA,
        ],
    )
    return f(numer2, perm3, s_arr)


def _make_mbias():
    r_ = jnp.arange(CHQ, dtype=jnp.int32)[:, None]
    c_ = jnp.arange(KW, dtype=jnp.int32)[None, :]
    qb = r_ // BL
    kb = c_ // BL
    band = (kb == qb) | (kb == qb + 1)
    selfm = c_ == r_ + BL
    return jnp.where(band & (~selfm), 0.0, -1e9).astype(jnp.float32)[None]


def kernel(query, value, rand_matrix):
    rm2 = rand_matrix.reshape(B, D, R * NH // 2)
    h, qv3 = _hash_stage(query, value, rm2)
    h2 = h.reshape(BR * L)
    perm, sh = _sort_stage(h2)
    perm3 = perm.reshape(BR, L // 128, 128)
    qvs = _gather_stage(qv3.reshape(B * L, 2 * D), perm3)
    sh_cur3 = sh.reshape(BR * (L // CHQ), 1, CHQ)
    sh_halo3 = sh.reshape(BR * NB, 1, BL)
    numer, s_arr = _attn_stage(qvs, sh_cur3, sh_halo3, _make_mbias())
    s2 = s_arr.reshape(BR * 128)
    out3 = _combine_stage(numer, perm3, s2)
    return out3.reshape(B, L, D)
